# baseline (speedup 1.0000x reference)
import jax
import jax.numpy as jnp
from jax import lax
from jax.experimental import pallas as pl
from jax.experimental.pallas import tpu as pltpu
from jax.experimental.pallas import tpu_sc as plsc

N = 10000
E = 160000
ANN = 256
HID = 64
D = ANN + HID
L = 8
OUT = 256

NC = 2
NS = 16
SPLIT = 5120
NPC = 5248
RPT = NPC // NS
CH = 32
GRP = 8
EGRAN = NS * CH * GRP
ECAP = 163840
PACK = 16384


def _sc_scatter_body(m, srcN, dstN, scal, zeros, p3,
                     sbuf, srcbuf, dstbuf, ra, rb, acc, ga, gb, sa):
    cid = lax.axis_index("c")
    sid = lax.axis_index("s")
    row0 = sid * RPT
    pltpu.sync_copy(zeros, acc.at[pl.ds(row0, RPT)])
    pltpu.sync_copy(scal, sbuf)
    sv = sbuf[...]
    gt = sv[0] * (1 - cid) + sv[1] * cid
    plsc.subcore_barrier()

    rows = [ra, rb]
    gsem = [ga, gb]

    def group(k, carry):
        base = (sid * gt + k) * GRP
        pltpu.sync_copy(srcN.at[cid, pl.ds(base, GRP)], srcbuf)
        pltpu.sync_copy(dstN.at[cid, pl.ds(base, GRP)], dstbuf)
        pltpu.async_copy(m.at[srcbuf.at[0]], ra, ga)
        for t in range(GRP):
            bt = rows[t % 2]
            st = gsem[t % 2]
            if t + 1 < GRP:
                pltpu.async_copy(m.at[srcbuf.at[t + 1]],
                                 rows[(t + 1) % 2], gsem[(t + 1) % 2])
            pltpu.make_async_copy(m.at[srcbuf.at[t]], bt, st).wait()
            pltpu.async_copy(bt, acc.at[dstbuf.at[t]], sa, add=True).wait()
        return carry

    lax.fori_loop(0, gt, group, 0, unroll=False)
    plsc.subcore_barrier()

    for k in range(RPT // CH):
        pltpu.sync_copy(acc.at[pl.ds(row0 + k * CH, CH)], ra)
        pltpu.sync_copy(ra, p3.at[cid, pl.ds(row0 + k * CH, CH)])
    rem = RPT - (RPT // CH) * CH
    pltpu.sync_copy(acc.at[pl.ds(row0 + RPT - rem, rem)], ra.at[pl.ds(0, rem)])
    pltpu.sync_copy(ra.at[pl.ds(0, rem)], p3.at[cid, pl.ds(row0 + RPT - rem, rem)])


_SC_CACHE = {}


def _sc_scatter(m, srcN, dstN, scal, zeros):
    fn = _SC_CACHE.get("k")
    if fn is None:
        fn = pl.kernel(
            _sc_scatter_body,
            out_type=jax.ShapeDtypeStruct((NC, NPC, D), jnp.float32),
            mesh=plsc.VectorSubcoreMesh(core_axis_name="c",
                                        subcore_axis_name="s"),
            scratch_types=[
                pltpu.VMEM((16,), jnp.int32),
                pltpu.VMEM((GRP, CH), jnp.int32),
                pltpu.VMEM((GRP, CH), jnp.int32),
                pltpu.VMEM((CH, D), jnp.float32),
                pltpu.VMEM((CH, D), jnp.float32),
                pltpu.VMEM_SHARED((NPC, D), jnp.float32),
                pltpu.SemaphoreType.DMA,
                pltpu.SemaphoreType.DMA,
                pltpu.SemaphoreType.DMA,
            ],
            compiler_params=pltpu.CompilerParams(use_tc_tiling_on_sc=False),
        )
        _SC_CACHE["k"] = fn
    return fn(m, srcN, dstN, scal, zeros)


BN = 1000
_MM = (((1,), (0,)), ((), ()))
_MT = (((1,), (1,)), ((), ()))


def _m0_body(h_ref, w_ref, m_ref):
    m_ref[...] = lax.dot_general(h_ref[...], w_ref[...], _MM,
                                 preferred_element_type=jnp.float32)


def _m0(h, w):
    return pl.pallas_call(
        _m0_body,
        grid=(N // BN,),
        in_specs=[
            pl.BlockSpec((BN, D), lambda i: (i, 0)),
            pl.BlockSpec((D, D), lambda i: (0, 0)),
        ],
        out_specs=pl.BlockSpec((BN, D), lambda i: (i, 0)),
        out_shape=jax.ShapeDtypeStruct((N, D), jnp.float32),
    )(h, w)


def _gh_body(h_ref, whr_ref, whz_ref, whn_ref, bh_ref,
             ghr_ref, ghz_ref, ghn_ref):
    h = h_ref[...]
    f32 = jnp.float32
    ghr_ref[...] = (lax.dot_general(h, whr_ref[...], _MT,
                                    preferred_element_type=f32)
                    + bh_ref[0, :D][None, :])
    ghz_ref[...] = (lax.dot_general(h, whz_ref[...], _MT,
                                    preferred_element_type=f32)
                    + bh_ref[0, D:2 * D][None, :])
    ghn_ref[...] = (lax.dot_general(h, whn_ref[...], _MT,
                                    preferred_element_type=f32)
                    + bh_ref[0, 2 * D:][None, :])


def _gh_layer(h, whh, b_hh2):
    wspec = pl.BlockSpec((D, D), lambda i: (0, 0))
    return pl.pallas_call(
        _gh_body,
        grid=(N // BN,),
        in_specs=[
            pl.BlockSpec((BN, D), lambda i: (i, 0)),
            wspec, wspec, wspec,
            pl.BlockSpec((1, 3 * D), lambda i: (0, 0)),
        ],
        out_specs=[pl.BlockSpec((BN, D), lambda i: (i, 0))] * 3,
        out_shape=[jax.ShapeDtypeStruct((N, D), jnp.float32)] * 3,
    )(h, whh[0], whh[1], whh[2], b_hh2)


def _gru_body(h_ref, agg_ref, ghr_ref, ghz_ref, ghn_ref,
              wir_ref, wiz_ref, win_ref,
              bi_ref, wnext_ref, out_ref, mn_ref):
    h = h_ref[...]
    agg = agg_ref[...]
    f32 = jnp.float32

    gi_r = (lax.dot_general(agg, wir_ref[...], _MT, preferred_element_type=f32)
            + bi_ref[0, :D][None, :])
    gi_z = (lax.dot_general(agg, wiz_ref[...], _MT, preferred_element_type=f32)
            + bi_ref[0, D:2 * D][None, :])
    gi_n = (lax.dot_general(agg, win_ref[...], _MT, preferred_element_type=f32)
            + bi_ref[0, 2 * D:][None, :])
    r = jax.nn.sigmoid(gi_r + ghr_ref[...])
    z = jax.nn.sigmoid(gi_z + ghz_ref[...])
    n = jnp.tanh(gi_n + r * ghn_ref[...])
    hn = (1.0 - z) * n + z * h
    out_ref[...] = hn
    mn_ref[...] = lax.dot_general(hn, wnext_ref[...], _MM,
                                  preferred_element_type=f32)


def _gru_layer(h, agg, gh, wih, b_ih2, w_next):
    wspec = pl.BlockSpec((D, D), lambda i: (0, 0))
    nspec = pl.BlockSpec((BN, D), lambda i: (i, 0))
    return pl.pallas_call(
        _gru_body,
        grid=(N // BN,),
        in_specs=[
            nspec, nspec, nspec, nspec, nspec,
            wspec, wspec, wspec,
            pl.BlockSpec((1, 3 * D), lambda i: (0, 0)),
            wspec,
        ],
        out_specs=[nspec] * 2,
        out_shape=[jax.ShapeDtypeStruct((N, D), jnp.float32)] * 2,
    )(h, agg, gh[0], gh[1], gh[2],
      wih[0], wih[1], wih[2],
      b_ih2, w_next)


def _head_body(h_ref, x_ref, w1_ref, w2_ref, b_ref, out_ref):
    f32 = jnp.float32
    out_ref[...] = (
        lax.dot_general(h_ref[...], w1_ref[...], _MT, preferred_element_type=f32)
        + lax.dot_general(x_ref[...], w2_ref[...], _MT, preferred_element_type=f32)
        + b_ref[0][None, :])


def _head(h, x, w_out, b_out):
    return pl.pallas_call(
        _head_body,
        grid=(N // BN,),
        in_specs=[
            pl.BlockSpec((BN, D), lambda i: (i, 0)),
            pl.BlockSpec((BN, ANN), lambda i: (i, 0)),
            pl.BlockSpec((OUT, D), lambda i: (0, 0)),
            pl.BlockSpec((OUT, ANN), lambda i: (0, 0)),
            pl.BlockSpec((1, OUT), lambda i: (0, 0)),
        ],
        out_specs=pl.BlockSpec((BN, OUT), lambda i: (i, 0)),
        out_shape=jax.ShapeDtypeStruct((N, OUT), jnp.float32),
    )(h, x, w_out[:, :D], w_out[:, D:], b_out[None, :])


def kernel(x, edge_index, batch, ggnn_w, w_ih, w_hh, b_ih, b_hh, w_out, b_out):
    src = edge_index[0]
    dst = edge_index[1]
    bit = (dst >= SPLIT).astype(jnp.int32)
    n0 = E - jnp.sum(bit)
    n1 = E - n0
    c0 = jnp.cumsum(1 - bit)
    c1 = jnp.cumsum(bit)
    pos = jnp.where(bit == 0, c0 - 1, ECAP + c1 - 1)
    packed = src * PACK + dst
    init = jnp.concatenate([
        jnp.full((ECAP,), SPLIT, jnp.int32),
        jnp.full((ECAP,), 2 * SPLIT, jnp.int32),
    ])
    allp = init.at[pos].set(packed).reshape(NC, ECAP)
    srcN = (allp // PACK).reshape(NC, ECAP // CH, CH)
    dstl = allp % PACK - SPLIT * jnp.arange(NC, dtype=jnp.int32)[:, None]
    dstN = dstl.reshape(NC, ECAP // CH, CH)
    g0 = (n0 + EGRAN - 1) // EGRAN
    g1 = (n1 + EGRAN - 1) // EGRAN
    scal = jnp.zeros((16,), jnp.int32).at[0].set(g0).at[1].set(g1)
    zeros = jnp.zeros((RPT, D), jnp.float32)

    wih = (w_ih[:D], w_ih[D:2 * D], w_ih[2 * D:])
    whh = (w_hh[:D], w_hh[D:2 * D], w_hh[2 * D:])
    b_ih2 = b_ih[None, :]
    b_hh2 = b_hh[None, :]

    h = jnp.pad(x, ((0, 0), (0, D - ANN)))
    m = _m0(h, ggnn_w[0])
    for l in range(L):
        p3 = _sc_scatter(m, srcN, dstN, scal, zeros)
        gh = _gh_layer(h, whh, b_hh2)
        agg = jnp.concatenate([p3[0, :SPLIT], p3[1, :N - SPLIT]])
        w_next = ggnn_w[(l + 1) % L]
        h, m = _gru_layer(h, agg, gh, wih, b_ih2, w_next)
    return _head(h, x, w_out, b_out)


# speedup vs baseline: 1.4091x; 1.4091x over previous
import jax
import jax.numpy as jnp
from jax import lax
from jax.experimental import pallas as pl
from jax.experimental.pallas import tpu as pltpu
from jax.experimental.pallas import tpu_sc as plsc

N = 10000
E = 160000
ANN = 256
HID = 64
D = ANN + HID
L = 8
OUT = 256

NC = 2
NS = 16
SPLIT = 5120
NPC = 5248
RPT = NPC // NS
CH = 40
GRP = 4
EGRAN = NS * CH * GRP
ECAP = 163840
PACK = 16384


def _sc_scatter_body(m, srcN, dstN, scal, zeros, p3,
                     sbuf, srcbuf, dstbuf, ra, rb, acc, ga, gb, sa):
    cid = lax.axis_index("c")
    sid = lax.axis_index("s")
    row0 = sid * RPT
    pltpu.sync_copy(zeros, acc.at[pl.ds(row0, RPT)])
    pltpu.sync_copy(scal, sbuf)
    sv = sbuf[...]
    gt = sv[0] * (1 - cid) + sv[1] * cid
    plsc.subcore_barrier()

    rows = [ra, rb]
    gsem = [ga, gb]

    def group(k, carry):
        base = (sid * gt + k) * GRP
        pltpu.sync_copy(srcN.at[cid, pl.ds(base, GRP)], srcbuf)
        pltpu.sync_copy(dstN.at[cid, pl.ds(base, GRP)], dstbuf)
        pltpu.async_copy(m.at[srcbuf.at[0]], ra, ga)
        for t in range(GRP):
            bt = rows[t % 2]
            st = gsem[t % 2]
            if t + 1 < GRP:
                pltpu.async_copy(m.at[srcbuf.at[t + 1]],
                                 rows[(t + 1) % 2], gsem[(t + 1) % 2])
            pltpu.make_async_copy(m.at[srcbuf.at[t]], bt, st).wait()
            pltpu.async_copy(bt, acc.at[dstbuf.at[t]], sa, add=True).wait()
        return carry

    lax.fori_loop(0, gt, group, 0, unroll=False)
    plsc.subcore_barrier()

    for k in range(RPT // CH):
        pltpu.sync_copy(acc.at[pl.ds(row0 + k * CH, CH)], ra)
        pltpu.sync_copy(ra, p3.at[cid, pl.ds(row0 + k * CH, CH)])
    rem = RPT - (RPT // CH) * CH
    pltpu.sync_copy(acc.at[pl.ds(row0 + RPT - rem, rem)], ra.at[pl.ds(0, rem)])
    pltpu.sync_copy(ra.at[pl.ds(0, rem)], p3.at[cid, pl.ds(row0 + RPT - rem, rem)])


_SC_CACHE = {}


def _sc_scatter(m, srcN, dstN, scal, zeros):
    fn = _SC_CACHE.get("k")
    if fn is None:
        fn = pl.kernel(
            _sc_scatter_body,
            out_type=jax.ShapeDtypeStruct((NC, NPC, D), jnp.float32),
            mesh=plsc.VectorSubcoreMesh(core_axis_name="c",
                                        subcore_axis_name="s"),
            scratch_types=[
                pltpu.VMEM((16,), jnp.int32),
                pltpu.VMEM((GRP, CH), jnp.int32),
                pltpu.VMEM((GRP, CH), jnp.int32),
                pltpu.VMEM((CH, D), jnp.float32),
                pltpu.VMEM((CH, D), jnp.float32),
                pltpu.VMEM_SHARED((NPC, D), jnp.float32),
                pltpu.SemaphoreType.DMA,
                pltpu.SemaphoreType.DMA,
                pltpu.SemaphoreType.DMA,
            ],
            compiler_params=pltpu.CompilerParams(use_tc_tiling_on_sc=False),
        )
        _SC_CACHE["k"] = fn
    return fn(m, srcN, dstN, scal, zeros)


BN = 1000
_MM = (((1,), (0,)), ((), ()))
_MT = (((1,), (1,)), ((), ()))


def _m0_body(h_ref, w_ref, m_ref):
    m_ref[...] = lax.dot_general(h_ref[...], w_ref[...], _MM,
                                 preferred_element_type=jnp.float32)


def _m0(h, w):
    return pl.pallas_call(
        _m0_body,
        grid=(N // BN,),
        in_specs=[
            pl.BlockSpec((BN, D), lambda i: (i, 0)),
            pl.BlockSpec((D, D), lambda i: (0, 0)),
        ],
        out_specs=pl.BlockSpec((BN, D), lambda i: (i, 0)),
        out_shape=jax.ShapeDtypeStruct((N, D), jnp.float32),
    )(h, w)


def _gru_body(h_ref, agg_ref,
              wir_ref, wiz_ref, win_ref, whr_ref, whz_ref, whn_ref,
              bi_ref, bh_ref, wnext_ref, out_ref, mn_ref):
    h = h_ref[...]
    agg = agg_ref[...]
    f32 = jnp.float32

    gi_r = (lax.dot_general(agg, wir_ref[...], _MT, preferred_element_type=f32)
            + bi_ref[0, :D][None, :])
    gi_z = (lax.dot_general(agg, wiz_ref[...], _MT, preferred_element_type=f32)
            + bi_ref[0, D:2 * D][None, :])
    gi_n = (lax.dot_general(agg, win_ref[...], _MT, preferred_element_type=f32)
            + bi_ref[0, 2 * D:][None, :])
    gh_r = (lax.dot_general(h, whr_ref[...], _MT, preferred_element_type=f32)
            + bh_ref[0, :D][None, :])
    gh_z = (lax.dot_general(h, whz_ref[...], _MT, preferred_element_type=f32)
            + bh_ref[0, D:2 * D][None, :])
    gh_n = (lax.dot_general(h, whn_ref[...], _MT, preferred_element_type=f32)
            + bh_ref[0, 2 * D:][None, :])
    r = jax.nn.sigmoid(gi_r + gh_r)
    z = jax.nn.sigmoid(gi_z + gh_z)
    n = jnp.tanh(gi_n + r * gh_n)
    hn = (1.0 - z) * n + z * h
    out_ref[...] = hn
    mn_ref[...] = lax.dot_general(hn, wnext_ref[...], _MM,
                                  preferred_element_type=f32)


def _gru_layer(h, agg, wih, whh, b_ih2, b_hh2, w_next):
    wspec = pl.BlockSpec((D, D), lambda i: (0, 0))
    return pl.pallas_call(
        _gru_body,
        grid=(N // BN,),
        in_specs=[
            pl.BlockSpec((BN, D), lambda i: (i, 0)),
            pl.BlockSpec((BN, D), lambda i: (i, 0)),
            wspec, wspec, wspec, wspec, wspec, wspec,
            pl.BlockSpec((1, 3 * D), lambda i: (0, 0)),
            pl.BlockSpec((1, 3 * D), lambda i: (0, 0)),
            wspec,
        ],
        out_specs=[pl.BlockSpec((BN, D), lambda i: (i, 0))] * 2,
        out_shape=[jax.ShapeDtypeStruct((N, D), jnp.float32)] * 2,
    )(h, agg,
      wih[0], wih[1], wih[2], whh[0], whh[1], whh[2],
      b_ih2, b_hh2, w_next)


def _head_body(h_ref, x_ref, w1_ref, w2_ref, b_ref, out_ref):
    f32 = jnp.float32
    out_ref[...] = (
        lax.dot_general(h_ref[...], w1_ref[...], _MT, preferred_element_type=f32)
        + lax.dot_general(x_ref[...], w2_ref[...], _MT, preferred_element_type=f32)
        + b_ref[0][None, :])


def _head(h, x, w_out, b_out):
    return pl.pallas_call(
        _head_body,
        grid=(N // BN,),
        in_specs=[
            pl.BlockSpec((BN, D), lambda i: (i, 0)),
            pl.BlockSpec((BN, ANN), lambda i: (i, 0)),
            pl.BlockSpec((OUT, D), lambda i: (0, 0)),
            pl.BlockSpec((OUT, ANN), lambda i: (0, 0)),
            pl.BlockSpec((1, OUT), lambda i: (0, 0)),
        ],
        out_specs=pl.BlockSpec((BN, OUT), lambda i: (i, 0)),
        out_shape=jax.ShapeDtypeStruct((N, OUT), jnp.float32),
    )(h, x, w_out[:, :D], w_out[:, D:], b_out[None, :])


def kernel(x, edge_index, batch, ggnn_w, w_ih, w_hh, b_ih, b_hh, w_out, b_out):
    src = edge_index[0]
    dst = edge_index[1]
    bit = (dst >= SPLIT).astype(jnp.int32)
    n0 = E - jnp.sum(bit)
    n1 = E - n0
    c0 = jnp.cumsum(1 - bit)
    c1 = jnp.cumsum(bit)
    pos = jnp.where(bit == 0, c0 - 1, ECAP + c1 - 1)
    packed = src * PACK + dst
    init = jnp.concatenate([
        jnp.full((ECAP,), SPLIT, jnp.int32),
        jnp.full((ECAP,), 2 * SPLIT, jnp.int32),
    ])
    allp = init.at[pos].set(packed).reshape(NC, ECAP)
    srcN = (allp // PACK).reshape(NC, ECAP // CH, CH)
    dstl = allp % PACK - SPLIT * jnp.arange(NC, dtype=jnp.int32)[:, None]
    dstN = dstl.reshape(NC, ECAP // CH, CH)
    g0 = (n0 + EGRAN - 1) // EGRAN
    g1 = (n1 + EGRAN - 1) // EGRAN
    scal = jnp.zeros((16,), jnp.int32).at[0].set(g0).at[1].set(g1)
    zeros = jnp.zeros((RPT, D), jnp.float32)

    wih = (w_ih[:D], w_ih[D:2 * D], w_ih[2 * D:])
    whh = (w_hh[:D], w_hh[D:2 * D], w_hh[2 * D:])
    b_ih2 = b_ih[None, :]
    b_hh2 = b_hh[None, :]

    h = jnp.pad(x, ((0, 0), (0, D - ANN)))
    m = _m0(h, ggnn_w[0])
    for l in range(L):
        p3 = _sc_scatter(m, srcN, dstN, scal, zeros)
        agg = jnp.concatenate([p3[0, :SPLIT], p3[1, :N - SPLIT]])
        w_next = ggnn_w[(l + 1) % L]
        h, m = _gru_layer(h, agg, wih, whh, b_ih2, b_hh2, w_next)
    return _head(h, x, w_out, b_out)
